# Initial kernel scaffold; baseline (speedup 1.0000x reference)
import jax
import jax.numpy as jnp
from jax.experimental import pallas as pl


def kernel(x, edge_index, edge_weight):
    raise NotImplementedError("write your pallas kernel here")


# speedup vs baseline: 4.5377x; 4.5377x over previous
import functools

import jax
import jax.numpy as jnp
from jax import lax
from jax.experimental import pallas as pl
from jax.experimental.pallas import tpu as pltpu
from jax.experimental.pallas import tpu_sc as plsc

_C = 1.0
_MAX_NORM = 1e6
_EPS = 1e-15
_BALL_EPS = 4e-3
_HALF = 64

_NC = 2
_NS = 16
_NW = _NC * _NS
_L = 16
_K = 128


def _artanh(x):
    x = jnp.clip(x, -1.0 + 1e-7, 1.0 - 1e-7)
    return 0.5 * (jnp.log1p(x) - jnp.log1p(-x))


def _pre_body(x_ref, o_ref):
    v = x_ref[...]
    outs = []
    for lo in (0, _HALF):
        s = v[:, lo:lo + _HALF]
        n = jnp.maximum(jnp.sqrt(jnp.sum(s * s, axis=1, keepdims=True)), _EPS)
        outs.append(s * (_artanh(n) / n))
    o_ref[...] = jnp.concatenate(outs, axis=1)


def _post_body(p0_ref, p1_ref, o_ref):
    s = jnp.minimum(p0_ref[...] + p1_ref[...], _MAX_NORM)
    outs = []
    for lo in (0, _HALF):
        u = s[:, lo:lo + _HALF]
        n = jnp.maximum(jnp.sqrt(jnp.sum(u * u, axis=1, keepdims=True)), _EPS)
        y = u * (jnp.tanh(n) / n)
        yn = jnp.maximum(jnp.sqrt(jnp.sum(y * y, axis=1, keepdims=True)), _EPS)
        maxn = 1.0 - _BALL_EPS
        outs.append(jnp.where(yn > maxn, y / yn * maxn, y))
    o_ref[...] = jnp.concatenate(outs, axis=1)


@functools.partial(jax.jit, static_argnums=(1, 2, 3))
def _sc_spmm(args, n_pad, d, per_w):
    nch = per_w // _K
    rows_per_tile = n_pad // _NS
    nzb = rows_per_tile // _K
    mesh = plsc.VectorSubcoreMesh(core_axis_name="c", subcore_axis_name="s")

    @functools.partial(
        pl.kernel,
        out_type=jax.ShapeDtypeStruct((_NC, n_pad, d), jnp.float32),
        mesh=mesh,
        scratch_types=[
            pltpu.VMEM((per_w,), jnp.int32),
            pltpu.VMEM((nch, _K), jnp.int32),
            pltpu.VMEM((per_w,), jnp.float32),
            pltpu.VMEM((_K, d), jnp.float32),
            pltpu.VMEM_SHARED((n_pad, d), jnp.float32),
            pltpu.SemaphoreType.DMA,
        ],
    )
    def spmm(xt_hbm, cols_hbm, rows_hbm, w_hbm, out_hbm,
             cols_v, rows_v, w_v, gbuf, acc, sem):
        cid = lax.axis_index("c")
        sid = lax.axis_index("s")
        wid = sid * _NC + cid
        pltpu.sync_copy(cols_hbm.at[wid], cols_v)
        pltpu.sync_copy(rows_hbm.at[wid], rows_v)
        pltpu.sync_copy(w_hbm.at[wid], w_v)

        zv = jnp.zeros((_L,), jnp.float32)

        def _zrow(i, carry):
            for l in range(d // _L):
                gbuf[i, pl.ds(l * _L, _L)] = zv
            return carry

        lax.fori_loop(0, _K, _zrow, 0)
        for b in range(nzb):
            pltpu.sync_copy(
                gbuf, acc.at[pl.ds(sid * rows_per_tile + b * _K, _K)])
        plsc.subcore_barrier()

        def _chunk(ch, carry):
            pltpu.async_copy(
                xt_hbm.at[cols_v.at[pl.ds(ch * _K, _K)]], gbuf, sem).wait()

            def _group(g, c2):
                wvec = w_v[pl.ds(ch * _K + g * _L, _L)]
                for j in range(_L):
                    w = wvec[j]
                    row = g * _L + j
                    for l in range(d // _L):
                        sl = pl.ds(l * _L, _L)
                        gbuf[row, sl] = gbuf[row, sl] * w
                return c2

            lax.fori_loop(0, _K // _L, _group, 0)
            pltpu.sync_copy(gbuf, acc.at[rows_v.at[ch]], add=True)
            return carry

        lax.fori_loop(0, nch, _chunk, 0)
        plsc.subcore_barrier()
        for b in range(nzb):
            off = sid * rows_per_tile + b * _K
            pltpu.sync_copy(acc.at[pl.ds(off, _K)],
                            out_hbm.at[cid, pl.ds(off, _K)])

    return spmm(*args)


def kernel(x, edge_index, edge_weight):
    n, d = x.shape
    e = edge_weight.shape[0]
    per_w = -(-e // (_NW * _K)) * _K
    n_pad = -(-n // (_NS * _K)) * (_NS * _K)

    blk = 1000
    grid = n // blk
    xt = pl.pallas_call(
        _pre_body,
        grid=(grid,),
        in_specs=[pl.BlockSpec((blk, d), lambda i: (i, 0))],
        out_specs=pl.BlockSpec((blk, d), lambda i: (i, 0)),
        out_shape=jax.ShapeDtypeStruct((n, d), jnp.float32),
    )(x)

    pad = _NW * per_w - e
    colp = jnp.pad(edge_index[1], (0, pad)).reshape(_NW, per_w)
    rowp = jnp.pad(edge_index[0], (0, pad)).reshape(_NW, per_w // _K, _K)
    wp = jnp.pad(edge_weight, (0, pad)).reshape(_NW, per_w)

    partials = _sc_spmm((xt, colp, rowp, wp), n_pad, d, per_w)

    out = pl.pallas_call(
        _post_body,
        grid=(grid,),
        in_specs=[pl.BlockSpec((blk, d), lambda i: (i, 0)),
                  pl.BlockSpec((blk, d), lambda i: (i, 0))],
        out_specs=pl.BlockSpec((blk, d), lambda i: (i, 0)),
        out_shape=jax.ShapeDtypeStruct((n, d), jnp.float32),
    )(partials[0], partials[1])
    return out


# speedup vs baseline: 5.6807x; 1.2519x over previous
import functools

import jax
import jax.numpy as jnp
from jax import lax
from jax.experimental import pallas as pl
from jax.experimental.pallas import tpu as pltpu
from jax.experimental.pallas import tpu_sc as plsc

_C = 1.0
_MAX_NORM = 1e6
_EPS = 1e-15
_BALL_EPS = 4e-3
_HALF = 64

_NC = 2
_NS = 16
_NW = _NC * _NS
_L = 16
_K = 128


def _artanh(x):
    x = jnp.clip(x, -1.0 + 1e-7, 1.0 - 1e-7)
    return 0.5 * (jnp.log1p(x) - jnp.log1p(-x))


def _pre_body(x_ref, o_ref):
    v = x_ref[...]
    outs = []
    for lo in (0, _HALF):
        s = v[:, lo:lo + _HALF]
        n = jnp.maximum(jnp.sqrt(jnp.sum(s * s, axis=1, keepdims=True)), _EPS)
        outs.append(s * (_artanh(n) / n))
    o_ref[...] = jnp.concatenate(outs, axis=1)


def _post_body(p0_ref, p1_ref, o_ref):
    s = jnp.minimum(p0_ref[...] + p1_ref[...], _MAX_NORM)
    outs = []
    for lo in (0, _HALF):
        u = s[:, lo:lo + _HALF]
        n = jnp.maximum(jnp.sqrt(jnp.sum(u * u, axis=1, keepdims=True)), _EPS)
        y = u * (jnp.tanh(n) / n)
        yn = jnp.maximum(jnp.sqrt(jnp.sum(y * y, axis=1, keepdims=True)), _EPS)
        maxn = 1.0 - _BALL_EPS
        outs.append(jnp.where(yn > maxn, y / yn * maxn, y))
    o_ref[...] = jnp.concatenate(outs, axis=1)


@functools.partial(jax.jit, static_argnums=(1, 2, 3))
def _sc_spmm(args, n_pad, d, per_w):
    nch = per_w // _K
    rows_per_tile = n_pad // _NS
    nzb = rows_per_tile // _K
    mesh = plsc.VectorSubcoreMesh(core_axis_name="c", subcore_axis_name="s")

    @functools.partial(
        pl.kernel,
        out_type=jax.ShapeDtypeStruct((_NC, n_pad, d), jnp.float32),
        mesh=mesh,
        scratch_types=[
            pltpu.VMEM((2, 3, _K), jnp.int32),
            pltpu.VMEM((2, _K, d), jnp.float32),
            pltpu.VMEM_SHARED((n_pad, d), jnp.float32),
            (pltpu.SemaphoreType.DMA, pltpu.SemaphoreType.DMA),
            (pltpu.SemaphoreType.DMA, pltpu.SemaphoreType.DMA),
        ],
    )
    def spmm(xt_hbm, eslab_hbm, out_hbm, ebuf, gbuf, acc, sems, esems):
        cid = lax.axis_index("c")
        sid = lax.axis_index("s")
        wid = sid * _NC + cid

        zv = jnp.zeros((_L,), jnp.float32)

        def _zrow(i, carry):
            for l in range(d // _L):
                gbuf[0, i, pl.ds(l * _L, _L)] = zv
            return carry

        lax.fori_loop(0, _K, _zrow, 0)
        for b in range(nzb):
            pltpu.sync_copy(
                gbuf.at[0], acc.at[pl.ds(sid * rows_per_tile + b * _K, _K)])
        plsc.subcore_barrier()

        def _estart(ch, b):
            pltpu.async_copy(eslab_hbm.at[wid, ch], ebuf.at[b], esems[b])

        def _ewait(b):
            pltpu.make_async_copy(
                eslab_hbm.at[0, 0], ebuf.at[b], esems[b]).wait()

        def _gstart(ch, b):
            del ch
            pltpu.async_copy(
                xt_hbm.at[ebuf.at[b, 0]], gbuf.at[b], sems[b])

        def _gwait(b):
            pltpu.make_async_copy(
                xt_hbm.at[pl.ds(0, _K)], gbuf.at[b], sems[b]).wait()

        def _process(ch, b):
            del ch

            def _group(g, c2):
                wvec = lax.bitcast_convert_type(
                    ebuf[b, 2, pl.ds(g * _L, _L)], jnp.float32)
                for j in range(_L):
                    w = wvec[j]
                    row = g * _L + j
                    for l in range(d // _L):
                        sl = pl.ds(l * _L, _L)
                        gbuf[b, row, sl] = gbuf[b, row, sl] * w
                return c2

            lax.fori_loop(0, _K // _L, _group, 0)
            pltpu.sync_copy(gbuf.at[b], acc.at[ebuf.at[b, 1]], add=True)

        _estart(0, 0)
        _ewait(0)
        _gstart(0, 0)
        _estart(1, 1)
        _ewait(1)
        _gstart(1, 1)

        def _pair(p, carry):
            ch0 = 2 * p
            _gwait(0)
            _process(ch0, 0)
            _estart(ch0 + 2, 0)
            _ewait(0)
            _gstart(ch0 + 2, 0)
            _gwait(1)
            _process(ch0 + 1, 1)

            @pl.when(ch0 + 3 < nch)
            def _():
                _estart(ch0 + 3, 1)
                _ewait(1)
                _gstart(ch0 + 3, 1)

            return carry

        lax.fori_loop(0, (nch - 1) // 2, _pair, 0)
        _gwait(0)
        _process(nch - 1, 0)
        plsc.subcore_barrier()
        for b in range(nzb):
            off = sid * rows_per_tile + b * _K
            pltpu.sync_copy(acc.at[pl.ds(off, _K)],
                            out_hbm.at[cid, pl.ds(off, _K)])

    return spmm(*args)


def kernel(x, edge_index, edge_weight):
    n, d = x.shape
    e = edge_weight.shape[0]
    per_w = -(-e // (_NW * _K)) * _K
    if (per_w // _K) % 2 == 0:
        per_w += _K
    n_pad = -(-n // (_NS * _K)) * (_NS * _K)

    blk = 1000
    grid = n // blk
    xt = pl.pallas_call(
        _pre_body,
        grid=(grid,),
        in_specs=[pl.BlockSpec((blk, d), lambda i: (i, 0))],
        out_specs=pl.BlockSpec((blk, d), lambda i: (i, 0)),
        out_shape=jax.ShapeDtypeStruct((n, d), jnp.float32),
    )(x)

    pad = _NW * per_w - e
    nch = per_w // _K
    colp = jnp.pad(edge_index[1], (0, pad)).reshape(_NW, nch, _K)
    rowp = jnp.pad(edge_index[0], (0, pad)).reshape(_NW, nch, _K)
    wbits = lax.bitcast_convert_type(jnp.pad(edge_weight, (0, pad)),
                                     jnp.int32).reshape(_NW, nch, _K)
    eslab = jnp.stack([colp, rowp, wbits], axis=2)

    partials = _sc_spmm((xt, eslab), n_pad, d, per_w)

    out = pl.pallas_call(
        _post_body,
        grid=(grid,),
        in_specs=[pl.BlockSpec((blk, d), lambda i: (i, 0)),
                  pl.BlockSpec((blk, d), lambda i: (i, 0))],
        out_specs=pl.BlockSpec((blk, d), lambda i: (i, 0)),
        out_shape=jax.ShapeDtypeStruct((n, d), jnp.float32),
    )(partials[0], partials[1])
    return out
